# baseline (speedup 1.0000x reference)
import functools

import jax
import jax.numpy as jnp
from jax import lax
from jax.experimental import pallas as pl
from jax.experimental.pallas import tpu as pltpu
from jax.experimental.pallas import tpu_sc as plsc

_NC = 2
_NS = 16
_NW = _NC * _NS


@functools.partial(jax.jit, static_argnames=("rows_per_w",))
def _sc_gather(table, idx_flat, *, rows_per_w):
    s_pad = 56
    n_idx = idx_flat.shape[0]
    d = table.shape[1]
    mesh = plsc.VectorSubcoreMesh(core_axis_name="c", subcore_axis_name="s")

    @functools.partial(
        pl.kernel,
        mesh=mesh,
        out_type=jax.ShapeDtypeStruct((n_idx, d), jnp.float32),
        scratch_types=[
            pltpu.VMEM((rows_per_w * s_pad,), jnp.int32),
            pltpu.VMEM((2, s_pad, d), jnp.float32),
            pltpu.SemaphoreType.DMA,
            pltpu.SemaphoreType.DMA,
            pltpu.SemaphoreType.DMA,
            pltpu.SemaphoreType.DMA,
        ],
    )
    def k(table_hbm, idx_hbm, out_hbm, idx_v, rows_v, gsem0, gsem1, wsem0, wsem1):
        wid = lax.axis_index("s") * _NC + lax.axis_index("c")
        base = wid * rows_per_w * s_pad
        pltpu.sync_copy(idx_hbm.at[pl.ds(base, rows_per_w * s_pad)], idx_v)

        bufs = (rows_v.at[0], rows_v.at[1])
        gsems = (gsem0, gsem1)
        wsems = (wsem0, wsem1)

        def start_gather(i, b):
            pltpu.async_copy(
                table_hbm.at[idx_v.at[pl.ds(i * s_pad, s_pad)]], bufs[b], gsems[b]
            )

        def wait_gather(b):
            pltpu.make_async_copy(
                table_hbm.at[idx_v.at[pl.ds(0, s_pad)]], bufs[b], gsems[b]
            ).wait()

        def start_write(i, b):
            pltpu.async_copy(
                bufs[b], out_hbm.at[pl.ds(base + i * s_pad, s_pad)], wsems[b]
            )

        def wait_write(b):
            pltpu.make_async_copy(
                bufs[b], out_hbm.at[pl.ds(base, s_pad)], wsems[b]
            ).wait()

        start_gather(0, 0)
        start_gather(1, 1)

        @pl.loop(0, rows_per_w // 2 - 1)
        def _(j):
            i0 = 2 * j
            wait_gather(0)
            start_write(i0, 0)
            wait_gather(1)
            start_write(i0 + 1, 1)
            wait_write(0)
            start_gather(i0 + 2, 0)
            wait_write(1)
            start_gather(i0 + 3, 1)

        wait_gather(0)
        start_write(rows_per_w - 2, 0)
        wait_gather(1)
        start_write(rows_per_w - 1, 1)
        wait_write(0)
        wait_write(1)

    return k(table, idx_flat)


def kernel(inputs, kernel):
    table = kernel
    ids = inputs.astype(jnp.int32)
    n_rows, s = ids.shape
    d = table.shape[1]
    assert n_rows % (2 * _NW) == 0 and s <= 56
    ids_p = jnp.pad(ids, ((0, 0), (0, 56 - s))) if s != 56 else ids
    out = _sc_gather(table, ids_p.reshape(-1), rows_per_w=n_rows // _NW)
    return out.reshape(n_rows, 56, d)[:, :s, :]


# speedup vs baseline: 1.8545x; 1.8545x over previous
import functools

import jax
import jax.numpy as jnp
from jax import lax
from jax.experimental import pallas as pl
from jax.experimental.pallas import tpu as pltpu
from jax.experimental.pallas import tpu_sc as plsc

_NC = 2
_NS = 16
_NW = _NC * _NS
_CHUNK = 128


@functools.partial(jax.jit, static_argnames=("rows_per_w",))
def _sc_gather(table, idx_flat, *, rows_per_w):
    n_idx = idx_flat.shape[0]
    d = table.shape[1]
    n_chunks = rows_per_w // _CHUNK
    mesh = plsc.VectorSubcoreMesh(core_axis_name="c", subcore_axis_name="s")

    @functools.partial(
        pl.kernel,
        mesh=mesh,
        out_type=jax.ShapeDtypeStruct((n_idx, d), jnp.float32),
        scratch_types=[
            pltpu.VMEM((rows_per_w,), jnp.int32),
            pltpu.VMEM((_CHUNK, d), jnp.float32),
            pltpu.SemaphoreType.DMA,
        ],
    )
    def k(table_hbm, idx_hbm, out_hbm, idx_v, rows_v, sem):
        wid = lax.axis_index("s") * _NC + lax.axis_index("c")
        base = wid * rows_per_w
        pltpu.sync_copy(idx_hbm.at[pl.ds(base, rows_per_w)], idx_v)

        @pl.loop(0, n_chunks)
        def _(i):
            pltpu.async_copy(
                table_hbm.at[idx_v.at[pl.ds(i * _CHUNK, _CHUNK)]], rows_v, sem
            ).wait()
            pltpu.sync_copy(rows_v, out_hbm.at[pl.ds(base + i * _CHUNK, _CHUNK)])

    return k(table, idx_flat)


def kernel(inputs, kernel):
    table = kernel
    ids = inputs.astype(jnp.int32)
    n_rows, s = ids.shape
    d = table.shape[1]
    n = n_rows * s
    assert n % (_NW * _CHUNK) == 0
    out = _sc_gather(table, ids.reshape(-1), rows_per_w=n // _NW)
    return out.reshape(n_rows, s, d)
